# Initial kernel scaffold; baseline (speedup 1.0000x reference)
import jax
import jax.numpy as jnp
from jax.experimental import pallas as pl


def kernel(input_ids, token_table, position_table):
    raise NotImplementedError("write your pallas kernel here")


# speedup vs baseline: 2.0260x; 2.0260x over previous
import functools

import jax
import jax.numpy as jnp
from jax import lax
from jax.experimental import pallas as pl
from jax.experimental.pallas import tpu as pltpu
from jax.experimental.pallas import tpu_sc as plsc

NUM_CORES = 2
NUM_SUBCORES = 16
NUM_WORKERS = NUM_CORES * NUM_SUBCORES
LANES = 16
IDX_ROW = 100


@functools.partial(jax.jit, static_argnums=(3, 4))
def _embed(ids2d, token_table, pos_s, S, D):
    n_idx_rows = ids2d.shape[0]
    n_rows = n_idx_rows * IDX_ROW
    rows_per_w = n_rows // NUM_WORKERS
    idx_rows_per_w = n_idx_rows // NUM_WORKERS
    chunks_per_w = rows_per_w // S
    gathers_per_chunk = S // IDX_ROW
    vregs_per_row = D // LANES

    mesh = plsc.VectorSubcoreMesh(
        core_axis_name="c", subcore_axis_name="s")

    @functools.partial(
        pl.kernel,
        mesh=mesh,
        out_type=jax.ShapeDtypeStruct((n_rows, D), jnp.float32),
        scratch_types=[
            pltpu.VMEM((idx_rows_per_w, IDX_ROW), jnp.int32),
            pltpu.VMEM((S, D), jnp.float32),
            pltpu.VMEM((S, D), jnp.float32),
            pltpu.SemaphoreType.DMA,
        ],
    )
    def body(ids_hbm, ttab_hbm, ptab_hbm, out_hbm, idx_v, pos_v, rows_v, sem):
        wid = lax.axis_index("s") * NUM_CORES + lax.axis_index("c")
        idx_base = wid * idx_rows_per_w
        row_base = wid * rows_per_w

        pltpu.sync_copy(ids_hbm.at[pl.ds(idx_base, idx_rows_per_w)], idx_v)
        pltpu.sync_copy(ptab_hbm.at[pl.ds(0, S)], pos_v)

        def chunk_body(c, carry):
            for g in range(gathers_per_chunk):
                pltpu.async_copy(
                    ttab_hbm.at[idx_v.at[c * gathers_per_chunk + g]],
                    rows_v.at[pl.ds(g * IDX_ROW, IDX_ROW)],
                    sem,
                ).wait()

            def add_row(r, carry2):
                for j in range(vregs_per_row):
                    sl = pl.ds(j * LANES, LANES)
                    rows_v[r, sl] = rows_v[r, sl] + pos_v[r, sl]
                return carry2

            lax.fori_loop(0, S, add_row, 0, unroll=4)

            pltpu.sync_copy(
                rows_v, out_hbm.at[pl.ds(row_base + c * S, S)])
            return carry

        lax.fori_loop(0, chunks_per_w, chunk_body, 0)

    return body(ids2d, token_table, pos_s)


def kernel(input_ids, token_table, position_table):
    B, S = input_ids.shape
    D = token_table.shape[1]
    ids2d = input_ids.reshape(-1, IDX_ROW).astype(jnp.int32)
    pos_s = position_table[:S]
    out = _embed(ids2d, token_table, pos_s, S, D)
    return out.reshape(B, S, D)


# speedup vs baseline: 4.5882x; 2.2647x over previous
import functools

import jax
import jax.numpy as jnp
from jax import lax
from jax.experimental import pallas as pl
from jax.experimental.pallas import tpu as pltpu
from jax.experimental.pallas import tpu_sc as plsc

NUM_CORES = 2
NUM_SUBCORES = 16
NUM_WORKERS = NUM_CORES * NUM_SUBCORES
LANES = 16
CH = 80
NBUF = 5


@functools.partial(jax.jit, static_argnums=(3, 4))
def _embed(ids2d, token_table, pos_s, S, D):
    n_idx_rows = ids2d.shape[0]
    n_rows = n_idx_rows * CH
    rows_per_w = n_rows // NUM_WORKERS
    chunks_per_w = rows_per_w // CH
    n_outer = chunks_per_w // NBUF
    vregs_per_row = D // LANES
    pos_ext = S + CH
    pos_off = [(b * CH) % S for b in range(NBUF)]

    mesh = plsc.VectorSubcoreMesh(
        core_axis_name="c", subcore_axis_name="s")

    @functools.partial(
        pl.kernel,
        mesh=mesh,
        out_type=jax.ShapeDtypeStruct((n_rows, D), jnp.float32),
        scratch_types=[
            pltpu.VMEM((chunks_per_w, CH), jnp.int32),
            pltpu.VMEM((pos_ext, D), jnp.float32),
            pltpu.VMEM((NBUF, CH, D), jnp.float32),
            pltpu.SemaphoreType.DMA,
            pltpu.SemaphoreType.DMA,
            pltpu.SemaphoreType.DMA,
            pltpu.SemaphoreType.DMA,
            pltpu.SemaphoreType.DMA,
            pltpu.SemaphoreType.DMA,
            pltpu.SemaphoreType.DMA,
            pltpu.SemaphoreType.DMA,
            pltpu.SemaphoreType.DMA,
            pltpu.SemaphoreType.DMA,
        ],
    )
    def body(ids_hbm, ttab_hbm, ptab_hbm, out_hbm, idx_v, pos_v, bufs,
             g0, g1, g2, g3, g4, s0, s1, s2, s3, s4):
        gsems = [g0, g1, g2, g3, g4]
        ssems = [s0, s1, s2, s3, s4]
        wid = lax.axis_index("s") * NUM_CORES + lax.axis_index("c")
        idx_base = wid * chunks_per_w
        row_base = wid * rows_per_w

        pltpu.sync_copy(ids_hbm.at[pl.ds(idx_base, chunks_per_w)], idx_v)
        pltpu.sync_copy(ptab_hbm.at[pl.ds(0, S)], pos_v.at[pl.ds(0, S)])
        pltpu.sync_copy(ptab_hbm.at[pl.ds(0, CH)], pos_v.at[pl.ds(S, CH)])

        def start_gather(c, b):
            pltpu.async_copy(ttab_hbm.at[idx_v.at[c]], bufs.at[b], gsems[b])

        def do_chunk(c, b, wait_store, prefetch):
            b2 = (b + 2) % NBUF
            pltpu.make_async_copy(
                ttab_hbm.at[idx_v.at[c]], bufs.at[b], gsems[b]).wait()
            if wait_store:
                pltpu.make_async_copy(
                    bufs.at[b2],
                    out_hbm.at[pl.ds(row_base + (c - 3) * CH, CH)],
                    ssems[b2]).wait()
            if prefetch:
                start_gather(c + 2, b2)

            pb = pos_off[b]

            def add_row(r, carry):
                for j in range(vregs_per_row):
                    sl = pl.ds(j * LANES, LANES)
                    bufs[b, r, sl] = bufs[b, r, sl] + pos_v[pb + r, sl]
                return carry

            lax.fori_loop(0, CH, add_row, 0, unroll=4)

            pltpu.async_copy(
                bufs.at[b], out_hbm.at[pl.ds(row_base + c * CH, CH)],
                ssems[b])

        start_gather(0, 0)
        start_gather(1, 1)

        for b in range(NBUF):
            do_chunk(b, b, wait_store=(b >= 3), prefetch=True)

        def outer(cc, carry):
            for b in range(NBUF):
                do_chunk(cc * NBUF + b, b, wait_store=True, prefetch=True)
            return carry

        lax.fori_loop(1, n_outer - 1, outer, 0)

        cl = (n_outer - 1) * NBUF
        for b in range(NBUF):
            do_chunk(cl + b, b, wait_store=True, prefetch=(b < 3))

        for b in (2, 3, 4):
            pltpu.make_async_copy(
                bufs.at[b],
                out_hbm.at[pl.ds(row_base + (cl + b) * CH, CH)],
                ssems[b]).wait()

    return body(ids2d, token_table, pos_s)


def kernel(input_ids, token_table, position_table):
    B, S = input_ids.shape
    D = token_table.shape[1]
    ids2d = input_ids.reshape(-1, CH).astype(jnp.int32)
    pos_s = position_table[:S]
    out = _embed(ids2d, token_table, pos_s, S, D)
    return out.reshape(B, S, D)
